# baseline (speedup 1.0000x reference)
import functools

import jax
import jax.numpy as jnp
from jax import lax
from jax.experimental import pallas as pl
from jax.experimental.pallas import tpu as pltpu
from jax.experimental.pallas import tpu_sc as plsc

VOCAB = 100000
DIM = 128
SEQ = 50
SEQ_PAD = 56
NC = 2
NS = 16
NW = NC * NS
NPIECE = 2
BATCH_P = 4096 // NPIECE
BPW = BATCH_P // NW
NBUF = 8
NGROUP = BPW // NBUF


def _body(x_hbm, tbl_hbm, out_hbm, idx_v, rows_v, gsem, osem):
    wid = lax.axis_index("s") * NC + lax.axis_index("c")
    pltpu.sync_copy(x_hbm.at[wid], idx_v)

    def start_gather(b, buf):
        pltpu.async_copy(
            tbl_hbm.at[idx_v.at[b, pl.ds(0, SEQ)]],
            rows_v.at[buf, pl.ds(0, SEQ)], gsem.at[buf])

    def wait_gather(buf):
        pltpu.make_async_copy(
            tbl_hbm.at[idx_v.at[0, pl.ds(0, SEQ)]],
            rows_v.at[buf, pl.ds(0, SEQ)], gsem.at[buf]).wait()

    def start_out(b, buf):
        pltpu.async_copy(
            rows_v.at[buf, pl.ds(0, SEQ)], out_hbm.at[wid * BPW + b],
            osem.at[buf])

    def wait_out(buf):
        pltpu.make_async_copy(
            rows_v.at[buf, pl.ds(0, SEQ)], out_hbm.at[0], osem.at[buf]).wait()

    for buf in range(NBUF):
        start_gather(buf, buf)

    def group(g, carry):
        for buf in range(NBUF):
            wait_gather(buf)
            start_out(g * NBUF + buf, buf)
        for buf in range(NBUF):
            wait_out(buf)

            @pl.when(g + 1 < NGROUP)
            def _():
                start_gather((g + 1) * NBUF + buf, buf)

        return carry

    lax.fori_loop(0, NGROUP, group, 0)


@jax.jit
def _run(x_r, emb_weight):
    mesh = plsc.VectorSubcoreMesh(core_axis_name="c", subcore_axis_name="s")
    k = pl.kernel(
        _body,
        out_type=jax.ShapeDtypeStruct((BATCH_P, SEQ, DIM), jnp.float32),
        mesh=mesh,
        scratch_types=[
            pltpu.VMEM((BPW, SEQ), jnp.int32),
            pltpu.VMEM((NBUF, SEQ_PAD, DIM), jnp.float32),
            pltpu.SemaphoreType.DMA((NBUF,)),
            pltpu.SemaphoreType.DMA((NBUF,)),
        ],
    )
    out = jnp.empty((4096, SEQ, DIM), jnp.float32)
    for p in range(NPIECE):
        out = lax.dynamic_update_slice(
            out, k(x_r[p], emb_weight), (p * BATCH_P, 0, 0))
    return out


def kernel(x, emb_weight):
    b, s = x.shape
    x_r = x.astype(jnp.int32).reshape(NPIECE, NW, BPW, s)
    return _run(x_r, emb_weight)


# speedup vs baseline: 1.0955x; 1.0955x over previous
import functools

import jax
import jax.numpy as jnp
from jax import lax
from jax.experimental import pallas as pl
from jax.experimental.pallas import tpu as pltpu
from jax.experimental.pallas import tpu_sc as plsc

VOCAB = 100000
DIM = 128
SEQ = 50
SEQ_PAD = 56
NC = 2
NS = 16
NW = NC * NS
NPIECE = 2
BATCH = 4096
BATCH_P = BATCH // NPIECE
BPW = BATCH_P // NW
NBUF = 8
NGROUP = BPW // NBUF
BB = 64


def _body(x_hbm, tbl_hbm, out_hbm, idx_v, rows_v, gsem, osem):
    wid = lax.axis_index("s") * NC + lax.axis_index("c")
    pltpu.sync_copy(x_hbm.at[wid], idx_v)

    def start_gather(b, buf):
        pltpu.async_copy(
            tbl_hbm.at[idx_v.at[b, pl.ds(0, SEQ)]],
            rows_v.at[buf, pl.ds(0, SEQ)], gsem.at[buf])

    def wait_gather(buf):
        pltpu.make_async_copy(
            tbl_hbm.at[idx_v.at[0, pl.ds(0, SEQ)]],
            rows_v.at[buf, pl.ds(0, SEQ)], gsem.at[buf]).wait()

    def start_out(b, buf):
        pltpu.async_copy(
            rows_v.at[buf], out_hbm.at[wid * BPW + b], osem.at[buf])

    def wait_out(buf):
        pltpu.make_async_copy(
            rows_v.at[buf], out_hbm.at[0], osem.at[buf]).wait()

    for buf in range(NBUF):
        start_gather(buf, buf)

    def group(g, carry):
        for buf in range(NBUF):
            wait_gather(buf)
            start_out(g * NBUF + buf, buf)
        for buf in range(NBUF):
            wait_out(buf)

            @pl.when(g + 1 < NGROUP)
            def _():
                start_gather((g + 1) * NBUF + buf, buf)

        return carry

    lax.fori_loop(0, NGROUP, group, 0)


def _repack0_body(src_ref, out_ref):
    out_ref[...] = src_ref[:, :SEQ, :]


def _repack1_body(src_ref, prev_ref, out_ref):
    del prev_ref
    out_ref[...] = src_ref[:, :SEQ, :]


@jax.jit
def _run(x_r, emb_weight):
    mesh = plsc.VectorSubcoreMesh(core_axis_name="c", subcore_axis_name="s")
    gather_k = pl.kernel(
        _body,
        out_type=jax.ShapeDtypeStruct((BATCH_P, SEQ_PAD, DIM), jnp.float32),
        mesh=mesh,
        scratch_types=[
            pltpu.VMEM((BPW, SEQ), jnp.int32),
            pltpu.VMEM((NBUF, SEQ_PAD, DIM), jnp.float32),
            pltpu.SemaphoreType.DMA((NBUF,)),
            pltpu.SemaphoreType.DMA((NBUF,)),
        ],
    )
    pieces = [gather_k(x_r[p], emb_weight) for p in range(NPIECE)]

    out = pl.pallas_call(
        _repack0_body,
        out_shape=jax.ShapeDtypeStruct((BATCH, SEQ, DIM), jnp.float32),
        grid=(BATCH_P // BB,),
        in_specs=[pl.BlockSpec((BB, SEQ_PAD, DIM), lambda i: (i, 0, 0))],
        out_specs=pl.BlockSpec((BB, SEQ, DIM), lambda i: (i, 0, 0)),
    )(pieces[0])
    for p in range(1, NPIECE):
        off = p * (BATCH_P // BB)
        out = pl.pallas_call(
            _repack1_body,
            out_shape=jax.ShapeDtypeStruct((BATCH, SEQ, DIM), jnp.float32),
            grid=(BATCH_P // BB,),
            in_specs=[
                pl.BlockSpec((BB, SEQ_PAD, DIM), lambda i: (i, 0, 0)),
                pl.BlockSpec(memory_space=pl.ANY),
            ],
            out_specs=pl.BlockSpec(
                (BB, SEQ, DIM), lambda i, _o=off: (i + _o, 0, 0)),
            input_output_aliases={1: 0},
        )(pieces[p], out)
    return out


def kernel(x, emb_weight):
    b, s = x.shape
    x_r = x.astype(jnp.int32).reshape(NPIECE, NW, BPW, s)
    return _run(x_r, emb_weight)


# speedup vs baseline: 1.4199x; 1.2961x over previous
import functools

import jax
import jax.numpy as jnp
from jax import lax
from jax.experimental import pallas as pl
from jax.experimental.pallas import tpu as pltpu
from jax.experimental.pallas import tpu_sc as plsc

VOCAB = 100000
DIM = 128
SEQ = 50
SEQ_PAD = 56
NC = 2
NS = 16
NW = NC * NS
BPW = 4096 // NW
NBUF = 8
NGROUP = BPW // NBUF


def _body(x_hbm, tbl_hbm, out_hbm, idx_v, rows_v, gsem, osem):
    wid = lax.axis_index("s") * NC + lax.axis_index("c")
    pltpu.sync_copy(x_hbm.at[wid], idx_v)

    def start_gather(b, buf):
        pltpu.async_copy(
            tbl_hbm.at[idx_v.at[b, pl.ds(0, SEQ)]],
            rows_v.at[buf, pl.ds(0, SEQ)], gsem.at[buf])

    def wait_gather(buf):
        pltpu.make_async_copy(
            tbl_hbm.at[idx_v.at[0, pl.ds(0, SEQ)]],
            rows_v.at[buf, pl.ds(0, SEQ)], gsem.at[buf]).wait()

    def start_out(b, buf):
        pltpu.async_copy(
            rows_v.at[buf], out_hbm.at[wid * BPW + b], osem.at[buf])

    def wait_out(buf):
        pltpu.make_async_copy(
            rows_v.at[buf], out_hbm.at[0], osem.at[buf]).wait()

    for buf in range(NBUF):
        start_gather(buf, buf)

    def group(g, carry):
        for buf in range(NBUF):
            wait_gather(buf)
            start_out(g * NBUF + buf, buf)
        for buf in range(NBUF):
            wait_out(buf)

            @pl.when(g + 1 < NGROUP)
            def _():
                start_gather((g + 1) * NBUF + buf, buf)

        return carry

    lax.fori_loop(0, NGROUP, group, 0)


@jax.jit
def _run(x_r, emb_weight):
    mesh = plsc.VectorSubcoreMesh(core_axis_name="c", subcore_axis_name="s")
    k = pl.kernel(
        _body,
        out_type=jax.ShapeDtypeStruct((4096, SEQ_PAD, DIM), jnp.float32),
        mesh=mesh,
        scratch_types=[
            pltpu.VMEM((BPW, SEQ), jnp.int32),
            pltpu.VMEM((NBUF, SEQ_PAD, DIM), jnp.float32),
            pltpu.SemaphoreType.DMA((NBUF,)),
            pltpu.SemaphoreType.DMA((NBUF,)),
        ],
    )
    return k(x_r, emb_weight)[:, :SEQ, :]


def kernel(x, emb_weight):
    b, s = x.shape
    x_r = x.astype(jnp.int32).reshape(NW, BPW, s)
    return _run(x_r, emb_weight)


# speedup vs baseline: 1.6642x; 1.1720x over previous
import functools

import jax
import jax.numpy as jnp
from jax import lax
from jax.experimental import pallas as pl
from jax.experimental.pallas import tpu as pltpu
from jax.experimental.pallas import tpu_sc as plsc

VOCAB = 100000
DIM = 128
SEQ = 50
SEQ_PAD = 56
NC = 2
NS = 16
NW = NC * NS
BATCH = 4096
BPW = BATCH // NW
NBUF = 8
NGROUP = BPW // NBUF


def _body(x_hbm, tbl_hbm, out_hbm, idx_v, rows_v, gsem, osem):
    wid = lax.axis_index("s") * NC + lax.axis_index("c")
    pltpu.sync_copy(x_hbm.at[pl.ds(wid * BPW, BPW)], idx_v)

    def start_gather(b, buf):
        pltpu.async_copy(
            tbl_hbm.at[idx_v.at[b, pl.ds(0, SEQ)]],
            rows_v.at[buf, pl.ds(0, SEQ)], gsem.at[buf])

    def wait_gather(buf):
        pltpu.make_async_copy(
            tbl_hbm.at[idx_v.at[0, pl.ds(0, SEQ)]],
            rows_v.at[buf, pl.ds(0, SEQ)], gsem.at[buf]).wait()

    def start_out(b, buf):
        pltpu.async_copy(
            rows_v.at[buf, pl.ds(0, SEQ)], out_hbm.at[wid * BPW + b],
            osem.at[buf])

    def wait_out(buf):
        pltpu.make_async_copy(
            rows_v.at[buf, pl.ds(0, SEQ)], out_hbm.at[0], osem.at[buf]).wait()

    for buf in range(NBUF):
        start_gather(buf, buf)

    def group(g, carry):
        for buf in range(NBUF):
            wait_gather(buf)
            start_out(g * NBUF + buf, buf)
        for buf in range(NBUF):
            wait_out(buf)

            @pl.when(g + 1 < NGROUP)
            def _():
                start_gather((g + 1) * NBUF + buf, buf)

        return carry

    lax.fori_loop(0, NGROUP, group, 0)


@jax.jit
def _run(x, emb_weight):
    mesh = plsc.VectorSubcoreMesh(core_axis_name="c", subcore_axis_name="s")
    k = pl.kernel(
        _body,
        out_type=jax.ShapeDtypeStruct((BATCH, SEQ, DIM), jnp.float32),
        mesh=mesh,
        scratch_types=[
            pltpu.VMEM((BPW, SEQ), jnp.int32),
            pltpu.VMEM((NBUF, SEQ_PAD, DIM), jnp.float32),
            pltpu.SemaphoreType.DMA((NBUF,)),
            pltpu.SemaphoreType.DMA((NBUF,)),
        ],
    )
    return k(x, emb_weight)


def kernel(x, emb_weight):
    return _run(x.astype(jnp.int32), emb_weight)
